# baseline (speedup 1.0000x reference)
import jax
import jax.numpy as jnp
from jax.experimental import pallas as pl

KNB = 20


def _knn_idx(x, k):
    xt = jnp.transpose(x, (0, 2, 1))
    inner = jnp.matmul(xt, x)
    sq = jnp.sum(x * x, axis=1)
    neg_dist = 2.0 * inner - sq[:, :, None] - sq[:, None, :]
    return jax.lax.top_k(neg_dist, k)[1]


def _gather_nb(x, idx):
    xt = jnp.transpose(x, (0, 2, 1))
    nb = jax.vmap(lambda a, i: a[i])(xt, idx)
    return jnp.transpose(nb, (0, 3, 1, 2))


def _bn(x, gamma, beta, axes, pshape):
    mean = jnp.mean(x, axis=axes, keepdims=True)
    var = jnp.var(x, axis=axes, keepdims=True)
    xn = (x - mean) * jax.lax.rsqrt(var + 1e-5)
    return xn * gamma.reshape(pshape) + beta.reshape(pshape)


def _lrelu(x):
    return jnp.where(x >= 0, x, 0.2 * x)


def _conv_bn(feat, p):
    out = jnp.einsum('oc,bcnk->bonk', p['W'], feat)
    return _lrelu(_bn(out, p['gamma'], p['beta'], (0, 2, 3), (1, -1, 1, 1)))


def _gec_layer1(xyz, nrm, p, k):
    idx = _knn_idx(xyz, k)
    xyz_j = _gather_nb(xyz, idx)
    n_j = _gather_nb(nrm, idx)
    xyz_i = jnp.broadcast_to(xyz[:, :, :, None], xyz_j.shape)
    n_i = jnp.broadcast_to(nrm[:, :, :, None], n_j.shape)
    rel = xyz_j - xyz_i
    dist = jnp.sqrt(jnp.sum(rel * rel, axis=1, keepdims=True) + 1e-12)
    dotn = jnp.sum(n_i * n_j, axis=1, keepdims=True)
    feat = jnp.concatenate([xyz_i, rel, dist, n_i, n_j, dotn], axis=1)
    return _conv_bn(feat, p)


def _gec_dyn(x, p, k):
    idx = _knn_idx(x, k)
    x_j = _gather_nb(x, idx)
    x_i = jnp.broadcast_to(x[:, :, :, None], x_j.shape)
    feat = jnp.concatenate([x_i, x_j - x_i], axis=1)
    return _conv_bn(feat, p)


def _identity_pallas(x):
    def body(i_ref, o_ref):
        o_ref[...] = i_ref[...]
    return pl.pallas_call(
        body, out_shape=jax.ShapeDtypeStruct(x.shape, x.dtype))(x)


def kernel(x, n, params):
    g = params['gec']
    feat = _gec_layer1(x, n, g[0], KNB)
    resx = feat
    x1 = jnp.max(feat, axis=-1)
    feat = _gec_dyn(x1, g[1], KNB) + resx
    x2 = jnp.max(feat, axis=-1)
    feat = _gec_dyn(x2, g[2], KNB)
    resx = feat
    x3 = jnp.max(feat, axis=-1)
    feat = _gec_dyn(x3, g[3], KNB) + resx
    x4 = jnp.max(feat, axis=-1)
    feat = _gec_dyn(x4, g[4], KNB)
    resx = feat
    x5 = jnp.max(feat, axis=-1)
    feat = _gec_dyn(x5, g[5], KNB) + resx
    x6 = jnp.max(feat, axis=-1)
    feat = _gec_dyn(x6, g[6], KNB)
    resx = feat
    x7 = jnp.max(feat, axis=-1)
    feat = _gec_dyn(x7, g[7], KNB) + resx
    x8 = jnp.max(feat, axis=-1)
    cat = jnp.concatenate([x1, x2, x3, x4, x5, x6, x7, x8], axis=1)
    h = jnp.einsum('oc,bcn->bon', params['conv4_W'], cat)
    h = _lrelu(_bn(h, params['conv4_gamma'], params['conv4_beta'], (0, 2), (1, -1, 1)))
    p1 = jnp.max(h, axis=-1)
    p2 = jnp.mean(h, axis=-1)
    z = jnp.concatenate([p1, p2], axis=1)
    z = _lrelu(_bn(z @ params['lin1_W'].T, params['bn6_gamma'], params['bn6_beta'], (0,), (1, -1)))
    z = _lrelu(_bn(z @ params['lin2_W'].T + params['lin2_b'], params['bn7_gamma'], params['bn7_beta'], (0,), (1, -1)))
    out = z @ params['lin3_W'].T + params['lin3_b']
    return _identity_pallas(out)


# speedup vs baseline: 1.2504x; 1.2504x over previous
import functools

import jax
import jax.numpy as jnp
from jax.experimental import pallas as pl
from jax.experimental.pallas import tpu as pltpu

KNB = 20
NPTS = 1024


def _knn_body(f_ref, idx_ref, nd_ref):
    f = f_ref[0]
    inner = jax.lax.dot_general(f, f, (((1,), (1,)), ((), ())),
                                preferred_element_type=jnp.float32)
    sq = jnp.sum(f * f, axis=1)
    nd_ref[...] = 2.0 * inner - sq[:, None] - sq[None, :]
    col = jax.lax.broadcasted_iota(jnp.int32, (NPTS, NPTS), 1)
    for t in range(KNB):
        nd = nd_ref[...]
        m = jnp.max(nd, axis=1)
        am = jnp.min(jnp.where(nd == m[:, None], col, NPTS), axis=1)
        idx_ref[0, t, :] = am
        nd_ref[...] = jnp.where(col == am[:, None], -jnp.inf, nd)


def _knn_topk(feats):
    B, N, C = feats.shape
    return pl.pallas_call(
        _knn_body,
        grid=(B,),
        in_specs=[pl.BlockSpec((1, N, C), lambda b: (b, 0, 0))],
        out_specs=pl.BlockSpec((1, KNB, N), lambda b: (b, 0, 0)),
        out_shape=jax.ShapeDtypeStruct((B, KNB, N), jnp.int32),
        scratch_shapes=[pltpu.VMEM((N, N), jnp.float32)],
    )(feats)


def _knn_idx(x):
    feats = jnp.transpose(x, (0, 2, 1))
    idx = _knn_topk(feats)
    return jnp.transpose(idx, (0, 2, 1))


def _gather_nb(x, idx):
    xt = jnp.transpose(x, (0, 2, 1))
    nb = jax.vmap(lambda a, i: a[i])(xt, idx)
    return jnp.transpose(nb, (0, 3, 1, 2))


def _bn(x, gamma, beta, axes, pshape):
    mean = jnp.mean(x, axis=axes, keepdims=True)
    var = jnp.var(x, axis=axes, keepdims=True)
    xn = (x - mean) * jax.lax.rsqrt(var + 1e-5)
    return xn * gamma.reshape(pshape) + beta.reshape(pshape)


def _lrelu(x):
    return jnp.where(x >= 0, x, 0.2 * x)


def _conv_bn(feat, p):
    out = jnp.einsum('oc,bcnk->bonk', p['W'], feat)
    return _lrelu(_bn(out, p['gamma'], p['beta'], (0, 2, 3), (1, -1, 1, 1)))


def _gec_layer1(xyz, nrm, p):
    idx = _knn_idx(xyz)
    xyz_j = _gather_nb(xyz, idx)
    n_j = _gather_nb(nrm, idx)
    xyz_i = jnp.broadcast_to(xyz[:, :, :, None], xyz_j.shape)
    n_i = jnp.broadcast_to(nrm[:, :, :, None], n_j.shape)
    rel = xyz_j - xyz_i
    dist = jnp.sqrt(jnp.sum(rel * rel, axis=1, keepdims=True) + 1e-12)
    dotn = jnp.sum(n_i * n_j, axis=1, keepdims=True)
    feat = jnp.concatenate([xyz_i, rel, dist, n_i, n_j, dotn], axis=1)
    return _conv_bn(feat, p)


def _gec_dyn(x, p):
    idx = _knn_idx(x)
    x_j = _gather_nb(x, idx)
    x_i = jnp.broadcast_to(x[:, :, :, None], x_j.shape)
    feat = jnp.concatenate([x_i, x_j - x_i], axis=1)
    return _conv_bn(feat, p)


def kernel(x, n, params):
    g = params['gec']
    feat = _gec_layer1(x, n, g[0])
    resx = feat
    x1 = jnp.max(feat, axis=-1)
    feat = _gec_dyn(x1, g[1]) + resx
    x2 = jnp.max(feat, axis=-1)
    feat = _gec_dyn(x2, g[2])
    resx = feat
    x3 = jnp.max(feat, axis=-1)
    feat = _gec_dyn(x3, g[3]) + resx
    x4 = jnp.max(feat, axis=-1)
    feat = _gec_dyn(x4, g[4])
    resx = feat
    x5 = jnp.max(feat, axis=-1)
    feat = _gec_dyn(x5, g[5]) + resx
    x6 = jnp.max(feat, axis=-1)
    feat = _gec_dyn(x6, g[6])
    resx = feat
    x7 = jnp.max(feat, axis=-1)
    feat = _gec_dyn(x7, g[7]) + resx
    x8 = jnp.max(feat, axis=-1)
    cat = jnp.concatenate([x1, x2, x3, x4, x5, x6, x7, x8], axis=1)
    h = jnp.einsum('oc,bcn->bon', params['conv4_W'], cat)
    h = _lrelu(_bn(h, params['conv4_gamma'], params['conv4_beta'], (0, 2), (1, -1, 1)))
    p1 = jnp.max(h, axis=-1)
    p2 = jnp.mean(h, axis=-1)
    z = jnp.concatenate([p1, p2], axis=1)
    z = _lrelu(_bn(z @ params['lin1_W'].T, params['bn6_gamma'], params['bn6_beta'], (0,), (1, -1)))
    z = _lrelu(_bn(z @ params['lin2_W'].T + params['lin2_b'], params['bn7_gamma'], params['bn7_beta'], (0,), (1, -1)))
    return z @ params['lin3_W'].T + params['lin3_b']
